# Initial kernel scaffold; baseline (speedup 1.0000x reference)
import jax
import jax.numpy as jnp
from jax.experimental import pallas as pl


def kernel(inputs):
    raise NotImplementedError("write your pallas kernel here")


# speedup vs baseline: 2.6989x; 2.6989x over previous
import functools

import jax
import jax.numpy as jnp
from jax import lax
from jax.experimental import pallas as pl
from jax.experimental.pallas import tpu as pltpu
from jax.experimental.pallas import tpu_sc as plsc

BATCH = 1024
LENGTH = 50
DIM = 32
BINS = DIM + DIM * DIM

NUM_CORES = 2
NUM_SUBCORES = 16
LANES = 16
NW = NUM_CORES * NUM_SUBCORES
ROWS_PER_W = BATCH // NW
GROUPS = ROWS_PER_W // LANES
PAIRS = LENGTH // 2


@functools.partial(
    pl.kernel,
    out_type=jax.ShapeDtypeStruct((BATCH * BINS,), jnp.float32),
    mesh=plsc.VectorSubcoreMesh(core_axis_name="c", subcore_axis_name="s"),
    scratch_types=[
        pltpu.VMEM((ROWS_PER_W * LENGTH,), jnp.int32),
        pltpu.VMEM((ROWS_PER_W * BINS,), jnp.float32),
    ],
    compiler_params=pltpu.CompilerParams(needs_layout_passes=False),
)
def _ngram_counts_sc(in_hbm, out_hbm, tok_v, cnt_v):
    wid = lax.axis_index("s") * NUM_CORES + lax.axis_index("c")
    row0 = wid * ROWS_PER_W

    pltpu.sync_copy(in_hbm.at[pl.ds(row0 * LENGTH, ROWS_PER_W * LENGTH)], tok_v)

    zeros = jnp.zeros((LANES,), jnp.float32)

    def _zero_body(i, carry):
        base = i * (8 * LANES)
        for j in range(8):
            cnt_v[pl.ds(base + j * LANES, LANES)] = zeros
        return carry

    lax.fori_loop(0, ROWS_PER_W * BINS // (8 * LANES), _zero_body, 0)

    lane = lax.iota(jnp.int32, 16)
    ones = jnp.ones((LANES,), jnp.float32)
    for g in range(GROUPS):
        tok_off = (g * LANES + lane) * LENGTH
        cnt_off = (g * LANES + lane) * BINS
        big_off = cnt_off + DIM
        for p in range(PAIRS):
            a = plsc.load_gather(tok_v, [tok_off + (2 * p)])
            b = plsc.load_gather(tok_v, [tok_off + (2 * p + 1)])
            plsc.addupdate_scatter(cnt_v, [cnt_off + a], ones)
            plsc.addupdate_scatter(cnt_v, [cnt_off + b], ones)
            plsc.addupdate_scatter(cnt_v, [big_off + a * DIM + b], ones)

    pltpu.sync_copy(cnt_v, out_hbm.at[pl.ds(row0 * BINS, ROWS_PER_W * BINS)])


def kernel(inputs):
    flat = inputs.reshape(BATCH * LENGTH)
    out = _ngram_counts_sc(flat)
    return out.reshape(BATCH, BINS)
